# baseline (speedup 1.0000x reference)
import functools

import jax
import jax.numpy as jnp
from jax.experimental import pallas as pl
from jax.experimental.pallas import tpu as pltpu
from jax.experimental.pallas import tpu_sc as plsc

_VOCAB = 1000000
_DIM = 32
_L = 16
_W = 128


def _sc_hash_gather_add(ids_f, prev_f, prev2_f, bw, tw, total):
    mesh = plsc.VectorSubcoreMesh(core_axis_name="c", subcore_axis_name="s")

    @functools.partial(
        pl.kernel,
        out_type=jax.ShapeDtypeStruct((total * _DIM,), jnp.float32),
        mesh=mesh,
        compiler_params=pltpu.CompilerParams(use_tc_tiling_on_sc=False),
        scratch_types=[
            pltpu.VMEM((_W,), jnp.int32),
            pltpu.VMEM((_W,), jnp.int32),
            pltpu.VMEM((_W, _DIM), jnp.float32),
            pltpu.VMEM((_W, _DIM), jnp.float32),
            pltpu.SemaphoreType.DMA,
            pltpu.SemaphoreType.DMA,
        ],
    )
    def k(ids_hbm, prev_hbm, prev2_hbm, bw_hbm, tw_hbm, out_hbm,
          bi_idx, tri_idx, rows_bi, rows_tri, s1, s2):

        def body(ids_v, prev_v, prev2_v, out_v):
            @pl.loop(0, _W, step=_L)
            def _(c):
                a = ids_v[pl.ds(c, _L)]
                p = prev_v[pl.ds(c, _L)]
                p2 = prev2_v[pl.ds(c, _L)]
                s = p * 131 + a
                bi_idx[pl.ds(c, _L)] = s % _VOCAB
                tri_idx[pl.ds(c, _L)] = (p2 * 173 + s) % _VOCAB

            c1 = pltpu.async_copy(bw_hbm.at[bi_idx], rows_bi, s1)
            c2 = pltpu.async_copy(tw_hbm.at[tri_idx], rows_tri, s2)
            c1.wait()
            c2.wait()

            @pl.loop(0, _W)
            def _(r):
                out_v[pl.ds(r * _DIM, _L)] = (
                    rows_bi[r, pl.ds(0, _L)] + rows_tri[r, pl.ds(0, _L)]
                )
                out_v[pl.ds(r * _DIM + _L, _L)] = (
                    rows_bi[r, pl.ds(_L, _L)] + rows_tri[r, pl.ds(_L, _L)]
                )

        pltpu.emit_pipeline(
            body,
            grid=(total // _W,),
            in_specs=[
                pl.BlockSpec((_W,), lambda i: (i,)),
                pl.BlockSpec((_W,), lambda i: (i,)),
                pl.BlockSpec((_W,), lambda i: (i,)),
            ],
            out_specs=[pl.BlockSpec((_W * _DIM,), lambda i: (i,))],
            core_axis_name=("c", "s"),
            dimension_semantics=(pltpu.PARALLEL,),
        )(ids_hbm, prev_hbm, prev2_hbm, out_hbm)

    return k(ids_f, prev_f, prev2_f, bw, tw)


def kernel(ids, bigram_weight, tri_weight):
    ids = ids.astype(jnp.int32)
    n, m = ids.shape
    total = n * m
    prev = jnp.zeros_like(ids).at[:, 1:].set(ids[:, :-1])
    prev2 = jnp.zeros_like(ids).at[:, 2:].set(ids[:, :-2])
    out = _sc_hash_gather_add(
        ids.reshape(total),
        prev.reshape(total),
        prev2.reshape(total),
        bigram_weight,
        tri_weight,
        total,
    )
    return out.reshape(n, m, _DIM)
